# baseline (speedup 1.0000x reference)
import functools

import jax
import jax.numpy as jnp
import numpy as np
from jax import lax
from jax.experimental import pallas as pl
from jax.experimental.pallas import tpu as pltpu
from jax.experimental.pallas import tpu_sc as plsc

N = 10000
NP = 10016
E = 160000
EP = 163840
T = 24
DH = 32
NW = 32
CPW = EP // NW // 128
EPW = EP // NW
RPT = NP // 16

BE = 2048
BN = 2504


def _expand8():
    m = np.zeros((8, 32), np.float32)
    for h in range(4):
        m[h, h * 8:(h + 1) * 8] = 1.0
    return jnp.asarray(m)


def _amat8(a):
    z = jnp.zeros((32, 8), jnp.float32)
    for h in range(4):
        z = z.at[h * 8:(h + 1) * 8, h].set(a[h])
    return z


def _mm(A, W, bm):
    M, K = A.shape
    D = W.shape[1]

    def body(a_ref, w_ref, o_ref):
        o_ref[...] = lax.dot_general(a_ref[...], w_ref[...],
                                     (((1,), (0,)), ((), ())),
                                     preferred_element_type=jnp.float32, precision=lax.Precision.HIGHEST)

    return pl.pallas_call(
        body,
        grid=(M // bm,),
        in_specs=[pl.BlockSpec((bm, K), lambda i: (i, 0)),
                  pl.BlockSpec((K, D), lambda i: (0, 0))],
        out_specs=pl.BlockSpec((bm, D), lambda i: (i, 0)),
        out_shape=jax.ShapeDtypeStruct((M, D), jnp.float32),
    )(A, W)


def _edge_math(S, R, EE, Amat, Exp, mask):
    def body(s_ref, r_ref, ee_ref, am_ref, ex_ref, mk_ref, wm_o, e8_o):
        u = s_ref[0] + r_ref[0] + ee_ref[...]
        u = jnp.where(u >= 0, u, 0.2 * u)
        logit = lax.dot_general(u, am_ref[...], (((1,), (0,)), ((), ())),
                                preferred_element_type=jnp.float32, precision=lax.Precision.HIGHEST)
        ex = jnp.exp(logit) * mk_ref[...]
        e8_o[0] = ex
        wm_o[0] = lax.dot_general(ex, ex_ref[...], (((1,), (0,)), ((), ())),
                                  preferred_element_type=jnp.float32, precision=lax.Precision.HIGHEST) * s_ref[0]

    return pl.pallas_call(
        body,
        grid=(T, EP // BE),
        in_specs=[
            pl.BlockSpec((1, BE, 32), lambda t, e: (t, e, 0)),
            pl.BlockSpec((1, BE, 32), lambda t, e: (t, e, 0)),
            pl.BlockSpec((BE, 32), lambda t, e: (e, 0)),
            pl.BlockSpec((32, 8), lambda t, e: (0, 0)),
            pl.BlockSpec((8, 32), lambda t, e: (0, 0)),
            pl.BlockSpec((1, 8), lambda t, e: (0, 0)),
        ],
        out_specs=[
            pl.BlockSpec((1, BE, 32), lambda t, e: (t, e, 0)),
            pl.BlockSpec((1, BE, 8), lambda t, e: (t, e, 0)),
        ],
        out_shape=[
            jax.ShapeDtypeStruct((T, EP, 32), jnp.float32),
            jax.ShapeDtypeStruct((T, EP, 8), jnp.float32),
        ],
    )(S, R, EE, Amat, Exp, mask)


def _finalize(n0, n1, d0, d1, Exp, b):
    def body(n0_r, n1_r, d0_r, d1_r, ex_r, b_r, o_r):
        num = n0_r[0] + n1_r[0]
        den = lax.dot_general(d0_r[0] + d1_r[0], ex_r[...],
                              (((1,), (0,)), ((), ())),
                              preferred_element_type=jnp.float32, precision=lax.Precision.HIGHEST) + 1e-16
        h = num / den + b_r[...]
        o_r[0] = jnp.where(h > 0, h, jnp.exp(h) - 1.0)

    return pl.pallas_call(
        body,
        grid=(T, NP // BN),
        in_specs=[
            pl.BlockSpec((1, BN, 32), lambda t, n: (t, n, 0)),
            pl.BlockSpec((1, BN, 32), lambda t, n: (t, n, 0)),
            pl.BlockSpec((1, BN, 8), lambda t, n: (t, n, 0)),
            pl.BlockSpec((1, BN, 8), lambda t, n: (t, n, 0)),
            pl.BlockSpec((8, 32), lambda t, n: (0, 0)),
            pl.BlockSpec((1, 32), lambda t, n: (0, 0)),
        ],
        out_specs=pl.BlockSpec((1, BN, 32), lambda t, n: (t, n, 0)),
        out_shape=jax.ShapeDtypeStruct((T, NP, 32), jnp.float32),
    )(n0, n1, d0, d1, Exp, b)


def _gen_einsum(Mp, H2):
    def body(m_ref, q_ref, o_ref):
        @pl.when(pl.program_id(1) == 0)
        def _():
            o_ref[...] = jnp.zeros_like(o_ref)

        o_ref[0] += lax.dot_general(m_ref[...], q_ref[0],
                                    (((0,), (0,)), ((), ())),
                                    preferred_element_type=jnp.float32, precision=lax.Precision.HIGHEST)

    return pl.pallas_call(
        body,
        grid=(T, NP // BN),
        in_specs=[
            pl.BlockSpec((BN, 128), lambda t, n: (n, 0)),
            pl.BlockSpec((1, BN, 32), lambda t, n: (t, n, 0)),
        ],
        out_specs=pl.BlockSpec((1, 128, 32), lambda t, n: (t, 0, 0)),
        out_shape=jax.ShapeDtypeStruct((T, 128, 32), jnp.float32),
    )(Mp, H2)


def _decoder(G, Wih_f, Whh_f, bf, Wih_b, Whh_b, bb, wof, wob, bo):
    def sig(x):
        return 1.0 / (1.0 + jnp.exp(-x))

    def body(g_r, wif_r, whf_r, bf_r, wib_r, whb_r, bb_r, wof_r, wob_r,
             bo_r, o_r):
        def cell(xt, h, c, wi, wh, bias):
            z = (lax.dot_general(xt, wi, (((1,), (1,)), ((), ())),
                                 preferred_element_type=jnp.float32, precision=lax.Precision.HIGHEST)
                 + lax.dot_general(h, wh, (((1,), (1,)), ((), ())),
                                   preferred_element_type=jnp.float32, precision=lax.Precision.HIGHEST)
                 + bias)
            i = z[:, 0:32]
            f = z[:, 32:64]
            g = z[:, 64:96]
            o = z[:, 96:128]
            c = sig(f) * c + sig(i) * jnp.tanh(g)
            h = sig(o) * jnp.tanh(c)
            return h, c

        zero = jnp.zeros((128, 32), jnp.float32)
        h, c = zero, zero
        hf = []
        for t in range(T):
            h, c = cell(g_r[t], h, c, wif_r[...], whf_r[...], bf_r[...])
            hf.append(h)
        h, c = zero, zero
        for t in range(T - 1, -1, -1):
            h, c = cell(g_r[t], h, c, wib_r[...], whb_r[...], bb_r[...])
            logit = (jnp.sum(hf[t] * wof_r[...], axis=1)
                     + jnp.sum(h * wob_r[...], axis=1) + bo_r[0, 0])
            o_r[t, :] = logit

    full = lambda s: pl.BlockSpec(s, lambda: tuple(0 for _ in s))
    return pl.pallas_call(
        body,
        in_specs=[
            full((T, 128, 32)),
            full((128, 32)), full((128, 32)), full((1, 128)),
            full((128, 32)), full((128, 32)), full((1, 128)),
            full((1, 32)), full((1, 32)), full((1, 1)),
        ],
        out_specs=full((T, 128)),
        out_shape=jax.ShapeDtypeStruct((T, 128), jnp.float32),
    )(G, Wih_f, Whh_f, bf, Wih_b, Whh_b, bb, wof, wob, bo)


def _sc_gather(XL, XR, rows_s, rows_d):
    S = XL[rows_s.reshape(-1)].reshape(T, EP, 32)
    R = XR[rows_d.reshape(-1)].reshape(T, EP, 32)
    return S, R


def _sc_scatter(wm, ex8, dstp):
    num = jax.vmap(lambda w: jax.ops.segment_sum(w, dstp, num_segments=NP))(wm)
    den = jax.vmap(lambda e: jax.ops.segment_sum(e, dstp, num_segments=NP))(ex8)
    z32 = jnp.zeros((T, NP, 32), jnp.float32)
    z8 = jnp.zeros((T, NP, 8), jnp.float32)
    return num, z32, den, z8


def kernel(x, edge_index, edge_attr, M, params):
    p = params
    Exp = _expand8()
    mask = jnp.asarray([[1.0] * 4 + [0.0] * 4], jnp.float32)

    srcp = jnp.concatenate([edge_index[0], jnp.zeros((EP - E,), jnp.int32)])
    dstp = jnp.concatenate([edge_index[1], jnp.full((EP - E,), N, jnp.int32)])
    ea = jnp.concatenate([edge_attr, jnp.zeros((EP - E, 3), jnp.float32)])

    toff = (jnp.arange(T, dtype=jnp.int32) * NP)[:, None]
    rows_s = (toff + srcp[None, :]).reshape(T, NW, CPW, 128)
    rows_d = (toff + dstp[None, :]).reshape(T, NW, CPW, 128)

    H = jnp.zeros((T, NP, 4), jnp.float32).at[:, :N, :].set(
        jnp.swapaxes(x, 0, 1))

    def layer(H, Wl, Wr, We, a, b):
        F = H.shape[-1]
        XL = _mm(H.reshape(T * NP, F), Wl, 5008)
        XR = _mm(H.reshape(T * NP, F), Wr, 5008)
        EE = _mm(ea, We, BE)
        S, R = _sc_gather(XL, XR, rows_s, rows_d)
        wm, ex8 = _edge_math(S, R, EE, _amat8(a), Exp, mask)
        n0, n1, d0, d1 = _sc_scatter(wm, ex8, dstp)
        return _finalize(n0, n1, d0, d1, Exp, b.reshape(1, 32))

    H1 = layer(H, p['Wl1'], p['Wr1'], p['We1'], p['a1'], p['b1'])
    H2 = layer(H1, p['Wl2'], p['Wr2'], p['We2'], p['a2'], p['b2'])

    Mp = jnp.zeros((NP, 128), jnp.float32).at[:N, :100].set(M)
    G = _gen_einsum(Mp, H2)
    Wo = p['Wo']
    logits = _decoder(G, p['Wih_f'], p['Whh_f'], p['bf'].reshape(1, 128),
                      p['Wih_b'], p['Whh_b'], p['bb'].reshape(1, 128),
                      Wo[:32, 0].reshape(1, 32), Wo[32:, 0].reshape(1, 32),
                      p['bo'].reshape(1, 1))
    return jnp.swapaxes(logits, 0, 1)[:100, :]


# speedup vs baseline: 9.4992x; 9.4992x over previous
import functools

import jax
import jax.numpy as jnp
import numpy as np
from jax import lax
from jax.experimental import pallas as pl
from jax.experimental.pallas import tpu as pltpu
from jax.experimental.pallas import tpu_sc as plsc

N = 10000
NP = 10016
E = 160000
EP = 163840
T = 24
DH = 32
NW = 32
CPW = EP // NW // 128
EPW = EP // NW
RPT = NP // 16

BE = 2048
BN = 2504


def _expand8():
    m = np.zeros((8, 32), np.float32)
    for h in range(4):
        m[h, h * 8:(h + 1) * 8] = 1.0
    return jnp.asarray(m)


def _amat8(a):
    z = jnp.zeros((32, 8), jnp.float32)
    for h in range(4):
        z = z.at[h * 8:(h + 1) * 8, h].set(a[h])
    return z


def _mm(A, W, bm):
    M, K = A.shape
    D = W.shape[1]

    def body(a_ref, w_ref, o_ref):
        o_ref[...] = lax.dot_general(a_ref[...], w_ref[...],
                                     (((1,), (0,)), ((), ())),
                                     preferred_element_type=jnp.float32, precision=lax.Precision.HIGHEST)

    return pl.pallas_call(
        body,
        grid=(M // bm,),
        in_specs=[pl.BlockSpec((bm, K), lambda i: (i, 0)),
                  pl.BlockSpec((K, D), lambda i: (0, 0))],
        out_specs=pl.BlockSpec((bm, D), lambda i: (i, 0)),
        out_shape=jax.ShapeDtypeStruct((M, D), jnp.float32),
    )(A, W)


def _edge_math(S, R, EE, Amat, Exp, mask):
    def body(s_ref, r_ref, ee_ref, am_ref, ex_ref, mk_ref, wm_o, e8_o):
        u = s_ref[0] + r_ref[0] + ee_ref[...]
        u = jnp.where(u >= 0, u, 0.2 * u)
        logit = lax.dot_general(u, am_ref[...], (((1,), (0,)), ((), ())),
                                preferred_element_type=jnp.float32, precision=lax.Precision.HIGHEST)
        ex = jnp.exp(logit) * mk_ref[...]
        e8_o[0] = ex
        wm_o[0] = lax.dot_general(ex, ex_ref[...], (((1,), (0,)), ((), ())),
                                  preferred_element_type=jnp.float32, precision=lax.Precision.HIGHEST) * s_ref[0]

    return pl.pallas_call(
        body,
        grid=(T, EP // BE),
        in_specs=[
            pl.BlockSpec((1, BE, 32), lambda t, e: (t, e, 0)),
            pl.BlockSpec((1, BE, 32), lambda t, e: (t, e, 0)),
            pl.BlockSpec((BE, 32), lambda t, e: (e, 0)),
            pl.BlockSpec((32, 8), lambda t, e: (0, 0)),
            pl.BlockSpec((8, 32), lambda t, e: (0, 0)),
            pl.BlockSpec((1, 8), lambda t, e: (0, 0)),
        ],
        out_specs=[
            pl.BlockSpec((1, BE, 32), lambda t, e: (t, e, 0)),
            pl.BlockSpec((1, BE, 8), lambda t, e: (t, e, 0)),
        ],
        out_shape=[
            jax.ShapeDtypeStruct((T, EP, 32), jnp.float32),
            jax.ShapeDtypeStruct((T, EP, 8), jnp.float32),
        ],
    )(S, R, EE, Amat, Exp, mask)


def _finalize(n0, n1, d0, d1, Exp, b):
    def body(n0_r, n1_r, d0_r, d1_r, ex_r, b_r, o_r):
        num = n0_r[0] + n1_r[0]
        den = lax.dot_general(d0_r[0] + d1_r[0], ex_r[...],
                              (((1,), (0,)), ((), ())),
                              preferred_element_type=jnp.float32, precision=lax.Precision.HIGHEST) + 1e-16
        h = num / den + b_r[...]
        o_r[0] = jnp.where(h > 0, h, jnp.exp(h) - 1.0)

    return pl.pallas_call(
        body,
        grid=(T, NP // BN),
        in_specs=[
            pl.BlockSpec((1, BN, 32), lambda t, n: (t, n, 0)),
            pl.BlockSpec((1, BN, 32), lambda t, n: (t, n, 0)),
            pl.BlockSpec((1, BN, 8), lambda t, n: (t, n, 0)),
            pl.BlockSpec((1, BN, 8), lambda t, n: (t, n, 0)),
            pl.BlockSpec((8, 32), lambda t, n: (0, 0)),
            pl.BlockSpec((1, 32), lambda t, n: (0, 0)),
        ],
        out_specs=pl.BlockSpec((1, BN, 32), lambda t, n: (t, n, 0)),
        out_shape=jax.ShapeDtypeStruct((T, NP, 32), jnp.float32),
    )(n0, n1, d0, d1, Exp, b)


def _gen_einsum(Mp, H2):
    def body(m_ref, q_ref, o_ref):
        @pl.when(pl.program_id(1) == 0)
        def _():
            o_ref[...] = jnp.zeros_like(o_ref)

        o_ref[0] += lax.dot_general(m_ref[...], q_ref[0],
                                    (((0,), (0,)), ((), ())),
                                    preferred_element_type=jnp.float32, precision=lax.Precision.HIGHEST)

    return pl.pallas_call(
        body,
        grid=(T, NP // BN),
        in_specs=[
            pl.BlockSpec((BN, 128), lambda t, n: (n, 0)),
            pl.BlockSpec((1, BN, 32), lambda t, n: (t, n, 0)),
        ],
        out_specs=pl.BlockSpec((1, 128, 32), lambda t, n: (t, 0, 0)),
        out_shape=jax.ShapeDtypeStruct((T, 128, 32), jnp.float32),
    )(Mp, H2)


def _decoder(G, Wih_f, Whh_f, bf, Wih_b, Whh_b, bb, wof, wob, bo):
    def sig(x):
        return 1.0 / (1.0 + jnp.exp(-x))

    def body(g_r, wif_r, whf_r, bf_r, wib_r, whb_r, bb_r, wof_r, wob_r,
             bo_r, o_r):
        def cell(xt, h, c, wi, wh, bias):
            z = (lax.dot_general(xt, wi, (((1,), (1,)), ((), ())),
                                 preferred_element_type=jnp.float32, precision=lax.Precision.HIGHEST)
                 + lax.dot_general(h, wh, (((1,), (1,)), ((), ())),
                                   preferred_element_type=jnp.float32, precision=lax.Precision.HIGHEST)
                 + bias)
            i = z[:, 0:32]
            f = z[:, 32:64]
            g = z[:, 64:96]
            o = z[:, 96:128]
            c = sig(f) * c + sig(i) * jnp.tanh(g)
            h = sig(o) * jnp.tanh(c)
            return h, c

        zero = jnp.zeros((128, 32), jnp.float32)
        h, c = zero, zero
        hf = []
        for t in range(T):
            h, c = cell(g_r[t], h, c, wif_r[...], whf_r[...], bf_r[...])
            hf.append(h)
        h, c = zero, zero
        for t in range(T - 1, -1, -1):
            h, c = cell(g_r[t], h, c, wib_r[...], whb_r[...], bb_r[...])
            logit = (jnp.sum(hf[t] * wof_r[...], axis=1)
                     + jnp.sum(h * wob_r[...], axis=1) + bo_r[0, 0])
            o_r[t, :] = logit

    full = lambda s: pl.BlockSpec(s, lambda: tuple(0 for _ in s))
    return pl.pallas_call(
        body,
        in_specs=[
            full((T, 128, 32)),
            full((128, 32)), full((128, 32)), full((1, 128)),
            full((128, 32)), full((128, 32)), full((1, 128)),
            full((1, 32)), full((1, 32)), full((1, 1)),
        ],
        out_specs=full((T, 128)),
        out_shape=jax.ShapeDtypeStruct((T, 128), jnp.float32),
    )(G, Wih_f, Whh_f, bf, Wih_b, Whh_b, bb, wof, wob, bo)


_MESH = plsc.VectorSubcoreMesh(core_axis_name="c", subcore_axis_name="s")


def _sc_gather(XL, XR, rows_s, rows_d):
    GC = 8
    GB = GC * 128

    @functools.partial(
        pl.kernel, mesh=_MESH,
        compiler_params=pltpu.CompilerParams(use_tc_tiling_on_sc=False),
        out_type=(jax.ShapeDtypeStruct((T, EP, 32), jnp.float32),
                  jax.ShapeDtypeStruct((T, EP, 32), jnp.float32)),
        scratch_types=[
            pltpu.VMEM((CPW, 128), jnp.int32),
            pltpu.VMEM((CPW, 128), jnp.int32),
            pltpu.VMEM((GB, 32), jnp.float32),
            pltpu.SemaphoreType.DMA,
        ],
    )
    def k(xl_hbm, xr_hbm, is_hbm, id_hbm, s_out, r_out, is_v, id_v, buf, sem):
        wid = lax.axis_index("s") * 2 + lax.axis_index("c")
        ebase = wid * EPW

        def per_t(t, carry):
            pltpu.sync_copy(is_hbm.at[t, wid], is_v)
            pltpu.sync_copy(id_hbm.at[t, wid], id_v)

            def make_pass(table, out, idxv):
                def grp(g, c2):
                    hs = [pltpu.async_copy(table.at[idxv.at[g * GC + j]],
                                           buf.at[pl.ds(j * 128, 128)], sem)
                          for j in range(GC)]
                    for h in hs:
                        h.wait()
                    pltpu.sync_copy(buf, out.at[t, pl.ds(ebase + g * GB, GB)])
                    return c2

                lax.fori_loop(0, CPW // GC, grp, 0)

            make_pass(xl_hbm, s_out, is_v)
            make_pass(xr_hbm, r_out, id_v)
            return carry

        lax.fori_loop(0, T, per_t, 0)

    return k(XL, XR, rows_s, rows_d)


def _sc_scatter(wm, ex8, dst_tiled, z32, z8):
    GC = 4
    GB = GC * 128

    @functools.partial(
        pl.kernel, mesh=_MESH,
        compiler_params=pltpu.CompilerParams(use_tc_tiling_on_sc=False),
        out_type=(jax.ShapeDtypeStruct((2 * T, NP, 32), jnp.float32),
                  jax.ShapeDtypeStruct((2 * T, NP, 8), jnp.float32)),
        scratch_types=[
            pltpu.VMEM((CPW, 128), jnp.int32),
            pltpu.VMEM((GB, 32), jnp.float32),
            pltpu.VMEM((GB, 8), jnp.float32),
            pltpu.VMEM_SHARED((NP, 32), jnp.float32),
            pltpu.VMEM_SHARED((NP, 8), jnp.float32),
            pltpu.SemaphoreType.DMA,
        ],
    )
    def k(wm_hbm, e8_hbm, dt_hbm, z32_hbm, z8_hbm, num_out, den_out,
          dv, wbuf, ebuf, acc_n, acc_d, sem):
        cid = lax.axis_index("c")
        sid = lax.axis_index("s")
        wid = sid * 2 + cid
        ebase = wid * EPW
        rbase = sid * RPT

        pltpu.sync_copy(dt_hbm.at[wid], dv)

        def per_t(t, carry):
            pltpu.sync_copy(z32_hbm, acc_n.at[pl.ds(rbase, RPT)])
            pltpu.sync_copy(z8_hbm, acc_d.at[pl.ds(rbase, RPT)])
            plsc.subcore_barrier()

            def grp(g, c2):
                h1 = pltpu.async_copy(
                    wm_hbm.at[t, pl.ds(ebase + g * GB, GB)], wbuf, sem)
                h2 = pltpu.async_copy(
                    e8_hbm.at[t, pl.ds(ebase + g * GB, GB)], ebuf, sem)
                h1.wait()
                h2.wait()
                for j in range(GC):
                    pltpu.sync_copy(wbuf.at[pl.ds(j * 128, 128)],
                                    acc_n.at[dv.at[g * GC + j]], add=True)
                    pltpu.sync_copy(ebuf.at[pl.ds(j * 128, 128)],
                                    acc_d.at[dv.at[g * GC + j]], add=True)
                return c2

            lax.fori_loop(0, CPW // GC, grp, 0)

            plsc.subcore_barrier()
            row = cid * T + t
            pltpu.sync_copy(acc_n.at[pl.ds(rbase, RPT)],
                            num_out.at[row, pl.ds(rbase, RPT)])
            pltpu.sync_copy(acc_d.at[pl.ds(rbase, RPT)],
                            den_out.at[row, pl.ds(rbase, RPT)])
            plsc.subcore_barrier()
            return carry

        lax.fori_loop(0, T, per_t, 0)

    num, den = k(wm, ex8, dst_tiled, z32, z8)
    num = num.reshape(2, T, NP, 32)
    den = den.reshape(2, T, NP, 8)
    return num[0], num[1], den[0], den[1]


def kernel(x, edge_index, edge_attr, M, params):
    p = params
    Exp = _expand8()
    mask = jnp.asarray([[1.0] * 4 + [0.0] * 4], jnp.float32)

    srcp = jnp.concatenate([edge_index[0], jnp.zeros((EP - E,), jnp.int32)])
    dstp = jnp.concatenate([edge_index[1], jnp.full((EP - E,), N, jnp.int32)])
    ea = jnp.concatenate([edge_attr, jnp.zeros((EP - E, 3), jnp.float32)])

    toff = (jnp.arange(T, dtype=jnp.int32) * NP)[:, None]
    rows_s = (toff + srcp[None, :]).reshape(T, NW, CPW, 128)
    rows_d = (toff + dstp[None, :]).reshape(T, NW, CPW, 128)
    dst_tiled = dstp.reshape(NW, CPW, 128)
    z32 = jnp.zeros((RPT, 32), jnp.float32)
    z8 = jnp.zeros((RPT, 8), jnp.float32)

    H = jnp.zeros((T, NP, 4), jnp.float32).at[:, :N, :].set(
        jnp.swapaxes(x, 0, 1))

    def layer(H, Wl, Wr, We, a, b):
        F = H.shape[-1]
        XL = _mm(H.reshape(T * NP, F), Wl, 5008)
        XR = _mm(H.reshape(T * NP, F), Wr, 5008)
        EE = _mm(ea, We, BE)
        S, R = _sc_gather(XL, XR, rows_s, rows_d)
        wm, ex8 = _edge_math(S, R, EE, _amat8(a), Exp, mask)
        n0, n1, d0, d1 = _sc_scatter(wm, ex8, dst_tiled, z32, z8)
        return _finalize(n0, n1, d0, d1, Exp, b.reshape(1, 32))

    H1 = layer(H, p['Wl1'], p['Wr1'], p['We1'], p['a1'], p['b1'])
    H2 = layer(H1, p['Wl2'], p['Wr2'], p['We2'], p['a2'], p['b2'])

    Mp = jnp.zeros((NP, 128), jnp.float32).at[:N, :100].set(M)
    G = _gen_einsum(Mp, H2)
    Wo = p['Wo']
    logits = _decoder(G, p['Wih_f'], p['Whh_f'], p['bf'].reshape(1, 128),
                      p['Wih_b'], p['Whh_b'], p['bb'].reshape(1, 128),
                      Wo[:32, 0].reshape(1, 32), Wo[32:, 0].reshape(1, 32),
                      p['bo'].reshape(1, 1))
    return jnp.swapaxes(logits, 0, 1)[:100, :]


# speedup vs baseline: 11.5536x; 1.2163x over previous
import functools

import jax
import jax.numpy as jnp
import numpy as np
from jax import lax
from jax.experimental import pallas as pl
from jax.experimental.pallas import tpu as pltpu
from jax.experimental.pallas import tpu_sc as plsc

N = 10000
NP = 10016
E = 160000
EP = 163840
T = 24
DH = 32
NW = 32
CPW = EP // NW // 128
EPW = EP // NW
RPT = NP // 16

BE = 2048
BN = 2504


def _expand8():
    m = np.zeros((8, 32), np.float32)
    for h in range(4):
        m[h, h * 8:(h + 1) * 8] = 1.0
    return jnp.asarray(m)


def _amat8(a):
    z = jnp.zeros((32, 8), jnp.float32)
    for h in range(4):
        z = z.at[h * 8:(h + 1) * 8, h].set(a[h])
    return z


def _mm(A, W, bm):
    M, K = A.shape
    D = W.shape[1]

    def body(a_ref, w_ref, o_ref):
        o_ref[...] = lax.dot_general(a_ref[...], w_ref[...],
                                     (((1,), (0,)), ((), ())),
                                     preferred_element_type=jnp.float32, precision=lax.Precision.HIGHEST)

    return pl.pallas_call(
        body,
        grid=(M // bm,),
        in_specs=[pl.BlockSpec((bm, K), lambda i: (i, 0)),
                  pl.BlockSpec((K, D), lambda i: (0, 0))],
        out_specs=pl.BlockSpec((bm, D), lambda i: (i, 0)),
        out_shape=jax.ShapeDtypeStruct((M, D), jnp.float32),
    )(A, W)


def _edge_math(S, R, EE, Amat, Exp, mask):
    def body(s_ref, r_ref, ee_ref, am_ref, ex_ref, mk_ref, wm_o, e8_o):
        u = s_ref[0] + r_ref[0] + ee_ref[...]
        u = jnp.where(u >= 0, u, 0.2 * u)
        logit = lax.dot_general(u, am_ref[...], (((1,), (0,)), ((), ())),
                                preferred_element_type=jnp.float32, precision=lax.Precision.HIGHEST)
        ex = jnp.exp(logit) * mk_ref[...]
        e8_o[0] = ex
        wm_o[0] = lax.dot_general(ex, ex_ref[...], (((1,), (0,)), ((), ())),
                                  preferred_element_type=jnp.float32, precision=lax.Precision.HIGHEST) * s_ref[0]

    return pl.pallas_call(
        body,
        grid=(T, EP // BE),
        in_specs=[
            pl.BlockSpec((1, BE, 32), lambda t, e: (t, e, 0)),
            pl.BlockSpec((1, BE, 32), lambda t, e: (t, e, 0)),
            pl.BlockSpec((BE, 32), lambda t, e: (e, 0)),
            pl.BlockSpec((32, 8), lambda t, e: (0, 0)),
            pl.BlockSpec((8, 32), lambda t, e: (0, 0)),
            pl.BlockSpec((1, 8), lambda t, e: (0, 0)),
        ],
        out_specs=[
            pl.BlockSpec((1, BE, 32), lambda t, e: (t, e, 0)),
            pl.BlockSpec((1, BE, 8), lambda t, e: (t, e, 0)),
        ],
        out_shape=[
            jax.ShapeDtypeStruct((T, EP, 32), jnp.float32),
            jax.ShapeDtypeStruct((T, EP, 8), jnp.float32),
        ],
    )(S, R, EE, Amat, Exp, mask)


def _finalize(n0, n1, d0, d1, Exp, b):
    def body(n0_r, n1_r, d0_r, d1_r, ex_r, b_r, o_r):
        num = n0_r[0] + n1_r[0]
        den = lax.dot_general(d0_r[0] + d1_r[0], ex_r[...],
                              (((1,), (0,)), ((), ())),
                              preferred_element_type=jnp.float32, precision=lax.Precision.HIGHEST) + 1e-16
        h = num / den + b_r[...]
        o_r[0] = jnp.where(h > 0, h, jnp.exp(h) - 1.0)

    return pl.pallas_call(
        body,
        grid=(T, NP // BN),
        in_specs=[
            pl.BlockSpec((1, BN, 32), lambda t, n: (t, n, 0)),
            pl.BlockSpec((1, BN, 32), lambda t, n: (t, n, 0)),
            pl.BlockSpec((1, BN, 8), lambda t, n: (t, n, 0)),
            pl.BlockSpec((1, BN, 8), lambda t, n: (t, n, 0)),
            pl.BlockSpec((8, 32), lambda t, n: (0, 0)),
            pl.BlockSpec((1, 32), lambda t, n: (0, 0)),
        ],
        out_specs=pl.BlockSpec((1, BN, 32), lambda t, n: (t, n, 0)),
        out_shape=jax.ShapeDtypeStruct((T, NP, 32), jnp.float32),
    )(n0, n1, d0, d1, Exp, b)


def _gen_einsum(Mp, H2):
    def body(m_ref, q_ref, o_ref):
        @pl.when(pl.program_id(1) == 0)
        def _():
            o_ref[...] = jnp.zeros_like(o_ref)

        o_ref[0] += lax.dot_general(m_ref[...], q_ref[0],
                                    (((0,), (0,)), ((), ())),
                                    preferred_element_type=jnp.float32, precision=lax.Precision.HIGHEST)

    return pl.pallas_call(
        body,
        grid=(T, NP // BN),
        in_specs=[
            pl.BlockSpec((BN, 128), lambda t, n: (n, 0)),
            pl.BlockSpec((1, BN, 32), lambda t, n: (t, n, 0)),
        ],
        out_specs=pl.BlockSpec((1, 128, 32), lambda t, n: (t, 0, 0)),
        out_shape=jax.ShapeDtypeStruct((T, 128, 32), jnp.float32),
    )(Mp, H2)


def _decoder(G, Wih_f, Whh_f, bf, Wih_b, Whh_b, bb, wof, wob, bo):
    def sig(x):
        return 1.0 / (1.0 + jnp.exp(-x))

    def body(g_r, wif_r, whf_r, bf_r, wib_r, whb_r, bb_r, wof_r, wob_r,
             bo_r, o_r):
        def cell(xt, h, c, wi, wh, bias):
            z = (lax.dot_general(xt, wi, (((1,), (1,)), ((), ())),
                                 preferred_element_type=jnp.float32, precision=lax.Precision.HIGHEST)
                 + lax.dot_general(h, wh, (((1,), (1,)), ((), ())),
                                   preferred_element_type=jnp.float32, precision=lax.Precision.HIGHEST)
                 + bias)
            i = z[:, 0:32]
            f = z[:, 32:64]
            g = z[:, 64:96]
            o = z[:, 96:128]
            c = sig(f) * c + sig(i) * jnp.tanh(g)
            h = sig(o) * jnp.tanh(c)
            return h, c

        zero = jnp.zeros((128, 32), jnp.float32)
        h, c = zero, zero
        hf = []
        for t in range(T):
            h, c = cell(g_r[t], h, c, wif_r[...], whf_r[...], bf_r[...])
            hf.append(h)
        h, c = zero, zero
        for t in range(T - 1, -1, -1):
            h, c = cell(g_r[t], h, c, wib_r[...], whb_r[...], bb_r[...])
            logit = (jnp.sum(hf[t] * wof_r[...], axis=1)
                     + jnp.sum(h * wob_r[...], axis=1) + bo_r[0, 0])
            o_r[t, :] = logit

    full = lambda s: pl.BlockSpec(s, lambda: tuple(0 for _ in s))
    return pl.pallas_call(
        body,
        in_specs=[
            full((T, 128, 32)),
            full((128, 32)), full((128, 32)), full((1, 128)),
            full((128, 32)), full((128, 32)), full((1, 128)),
            full((1, 32)), full((1, 32)), full((1, 1)),
        ],
        out_specs=full((T, 128)),
        out_shape=jax.ShapeDtypeStruct((T, 128), jnp.float32),
    )(G, Wih_f, Whh_f, bf, Wih_b, Whh_b, bb, wof, wob, bo)


_MESH = plsc.VectorSubcoreMesh(core_axis_name="c", subcore_axis_name="s")


def _sc_edge_layer(XL, XR, EE, rows_s, rows_d, dst_tiled, a_flat, z32, z8):
    @functools.partial(
        pl.kernel, mesh=_MESH,
        compiler_params=pltpu.CompilerParams(use_tc_tiling_on_sc=False,
                                             needs_layout_passes=False),
        out_type=(jax.ShapeDtypeStruct((2 * T, NP, 32), jnp.float32),
                  jax.ShapeDtypeStruct((2 * T, NP, 8), jnp.float32)),
        scratch_types=[
            pltpu.VMEM((CPW, 128), jnp.int32),
            pltpu.VMEM((CPW, 128), jnp.int32),
            pltpu.VMEM((CPW, 128), jnp.int32),
            pltpu.VMEM((32, 16), jnp.float32),
            pltpu.VMEM((128, 32), jnp.float32),
            pltpu.VMEM((128, 32), jnp.float32),
            pltpu.VMEM((128, 32), jnp.float32),
            pltpu.VMEM((128, 32), jnp.float32),
            pltpu.VMEM((128, 8), jnp.float32),
            pltpu.VMEM_SHARED((NP, 32), jnp.float32),
            pltpu.VMEM_SHARED((NP, 8), jnp.float32),
            pltpu.SemaphoreType.DMA,
        ],
    )
    def k(xl_hbm, xr_hbm, ee_hbm, is_hbm, id_hbm, dt_hbm, a_hbm, z32_hbm,
          z8_hbm, num_out, den_out,
          is_v, id_v, dv, a_v, sbuf, rbuf, ebuf, wmbuf, exbuf,
          acc_n, acc_d, sem):
        iota = lax.iota(jnp.int32, 16)
        cid = lax.axis_index("c")
        sid = lax.axis_index("s")
        wid = sid * 2 + cid
        ebase = wid * EPW
        rbase = sid * RPT

        pltpu.sync_copy(dt_hbm.at[wid], dv)
        pltpu.sync_copy(a_hbm, a_v)
        pltpu.sync_copy(z8_hbm.at[pl.ds(0, 128)], exbuf)

        def per_t(t, carry):
            pltpu.sync_copy(is_hbm.at[t, wid], is_v)
            pltpu.sync_copy(id_hbm.at[t, wid], id_v)
            pltpu.sync_copy(z32_hbm, acc_n.at[pl.ds(rbase, RPT)])
            pltpu.sync_copy(z8_hbm, acc_d.at[pl.ds(rbase, RPT)])
            plsc.subcore_barrier()

            def chunk(kk, c2):
                h1 = pltpu.async_copy(xl_hbm.at[is_v.at[kk]], sbuf, sem)
                h2 = pltpu.async_copy(xr_hbm.at[id_v.at[kk]], rbuf, sem)
                h3 = pltpu.async_copy(
                    ee_hbm.at[pl.ds(ebase + kk * 128, 128)], ebuf, sem)
                h1.wait()
                h2.wait()
                h3.wait()

                def exp32(xv):
                    nf = (xv * 1.4426950408889634 + 12582912.0) - 12582912.0
                    r = (xv - nf * 0.693359375) + nf * 2.1219444005469057e-4
                    p = 1.0 / 720.0 + r * 0.0
                    p = 1.0 / 120.0 + r * p
                    p = 1.0 / 24.0 + r * p
                    p = 1.0 / 6.0 + r * p
                    p = 0.5 + r * p
                    p = 1.0 + r * p
                    p = 1.0 + r * p
                    ni = lax.convert_element_type(nf, jnp.int32)
                    sc = lax.bitcast_convert_type(
                        lax.shift_left(ni + 127, 23), jnp.float32)
                    return p * sc

                def grp(g, c3):
                    jvec = iota + g * 16
                    acc = [jnp.zeros((16,), jnp.float32) for _ in range(4)]
                    for c in range(32):
                        cv = jnp.full((16,), c, jnp.int32)
                        u = (plsc.load_gather(sbuf, [jvec, cv])
                             + plsc.load_gather(rbuf, [jvec, cv])
                             + plsc.load_gather(ebuf, [jvec, cv]))
                        u = jnp.maximum(u, 0.0) + 0.2 * jnp.minimum(u, 0.0)
                        acc[c // 8] = acc[c // 8] + u * a_v[c, :]
                    for h in range(4):
                        hv = jnp.full((16,), h, jnp.int32)
                        exh = exp32(acc[h])
                        plsc.store_scatter(exbuf, [jvec, hv], exh)
                        for d in range(8):
                            c = h * 8 + d
                            cv = jnp.full((16,), c, jnp.int32)
                            wc = plsc.load_gather(sbuf, [jvec, cv]) * exh
                            plsc.store_scatter(wmbuf, [jvec, cv], wc)
                    return c3

                lax.fori_loop(0, 8, grp, 0)
                pltpu.sync_copy(wmbuf, acc_n.at[dv.at[kk]], add=True)
                pltpu.sync_copy(exbuf, acc_d.at[dv.at[kk]], add=True)
                return c2

            lax.fori_loop(0, CPW, chunk, 0)

            plsc.subcore_barrier()
            row = cid * T + t
            pltpu.sync_copy(acc_n.at[pl.ds(rbase, RPT)],
                            num_out.at[row, pl.ds(rbase, RPT)])
            pltpu.sync_copy(acc_d.at[pl.ds(rbase, RPT)],
                            den_out.at[row, pl.ds(rbase, RPT)])
            plsc.subcore_barrier()
            return carry

        lax.fori_loop(0, T, per_t, 0)

    num, den = k(XL, XR, EE, rows_s, rows_d, dst_tiled, a_flat, z32, z8)
    num = num.reshape(2, T, NP, 32)
    den = den.reshape(2, T, NP, 8)
    return num[0], num[1], den[0], den[1]


def _sc_gather(XL, XR, rows_s, rows_d):
    GC = 8
    GB = GC * 128

    @functools.partial(
        pl.kernel, mesh=_MESH,
        compiler_params=pltpu.CompilerParams(use_tc_tiling_on_sc=False),
        out_type=(jax.ShapeDtypeStruct((T, EP, 32), jnp.float32),
                  jax.ShapeDtypeStruct((T, EP, 32), jnp.float32)),
        scratch_types=[
            pltpu.VMEM((CPW, 128), jnp.int32),
            pltpu.VMEM((CPW, 128), jnp.int32),
            pltpu.VMEM((GB, 32), jnp.float32),
            pltpu.SemaphoreType.DMA,
        ],
    )
    def k(xl_hbm, xr_hbm, is_hbm, id_hbm, s_out, r_out, is_v, id_v, buf, sem):
        wid = lax.axis_index("s") * 2 + lax.axis_index("c")
        ebase = wid * EPW

        def per_t(t, carry):
            pltpu.sync_copy(is_hbm.at[t, wid], is_v)
            pltpu.sync_copy(id_hbm.at[t, wid], id_v)

            def make_pass(table, out, idxv):
                def grp(g, c2):
                    hs = [pltpu.async_copy(table.at[idxv.at[g * GC + j]],
                                           buf.at[pl.ds(j * 128, 128)], sem)
                          for j in range(GC)]
                    for h in hs:
                        h.wait()
                    pltpu.sync_copy(buf, out.at[t, pl.ds(ebase + g * GB, GB)])
                    return c2

                lax.fori_loop(0, CPW // GC, grp, 0)

            make_pass(xl_hbm, s_out, is_v)
            make_pass(xr_hbm, r_out, id_v)
            return carry

        lax.fori_loop(0, T, per_t, 0)

    return k(XL, XR, rows_s, rows_d)


def _sc_scatter(wm, ex8, dst_tiled, z32, z8):
    GC = 4
    GB = GC * 128

    @functools.partial(
        pl.kernel, mesh=_MESH,
        compiler_params=pltpu.CompilerParams(use_tc_tiling_on_sc=False),
        out_type=(jax.ShapeDtypeStruct((2 * T, NP, 32), jnp.float32),
                  jax.ShapeDtypeStruct((2 * T, NP, 8), jnp.float32)),
        scratch_types=[
            pltpu.VMEM((CPW, 128), jnp.int32),
            pltpu.VMEM((GB, 32), jnp.float32),
            pltpu.VMEM((GB, 8), jnp.float32),
            pltpu.VMEM_SHARED((NP, 32), jnp.float32),
            pltpu.VMEM_SHARED((NP, 8), jnp.float32),
            pltpu.SemaphoreType.DMA,
        ],
    )
    def k(wm_hbm, e8_hbm, dt_hbm, z32_hbm, z8_hbm, num_out, den_out,
          dv, wbuf, ebuf, acc_n, acc_d, sem):
        cid = lax.axis_index("c")
        sid = lax.axis_index("s")
        wid = sid * 2 + cid
        ebase = wid * EPW
        rbase = sid * RPT

        pltpu.sync_copy(dt_hbm.at[wid], dv)

        def per_t(t, carry):
            pltpu.sync_copy(z32_hbm, acc_n.at[pl.ds(rbase, RPT)])
            pltpu.sync_copy(z8_hbm, acc_d.at[pl.ds(rbase, RPT)])
            plsc.subcore_barrier()

            def grp(g, c2):
                h1 = pltpu.async_copy(
                    wm_hbm.at[t, pl.ds(ebase + g * GB, GB)], wbuf, sem)
                h2 = pltpu.async_copy(
                    e8_hbm.at[t, pl.ds(ebase + g * GB, GB)], ebuf, sem)
                h1.wait()
                h2.wait()
                for j in range(GC):
                    pltpu.sync_copy(wbuf.at[pl.ds(j * 128, 128)],
                                    acc_n.at[dv.at[g * GC + j]], add=True)
                    pltpu.sync_copy(ebuf.at[pl.ds(j * 128, 128)],
                                    acc_d.at[dv.at[g * GC + j]], add=True)
                return c2

            lax.fori_loop(0, CPW // GC, grp, 0)

            plsc.subcore_barrier()
            row = cid * T + t
            pltpu.sync_copy(acc_n.at[pl.ds(rbase, RPT)],
                            num_out.at[row, pl.ds(rbase, RPT)])
            pltpu.sync_copy(acc_d.at[pl.ds(rbase, RPT)],
                            den_out.at[row, pl.ds(rbase, RPT)])
            plsc.subcore_barrier()
            return carry

        lax.fori_loop(0, T, per_t, 0)

    num, den = k(wm, ex8, dst_tiled, z32, z8)
    num = num.reshape(2, T, NP, 32)
    den = den.reshape(2, T, NP, 8)
    return num[0], num[1], den[0], den[1]


def kernel(x, edge_index, edge_attr, M, params):
    p = params
    Exp = _expand8()
    mask = jnp.asarray([[1.0] * 4 + [0.0] * 4], jnp.float32)

    srcp = jnp.concatenate([edge_index[0], jnp.zeros((EP - E,), jnp.int32)])
    dstp = jnp.concatenate([edge_index[1], jnp.full((EP - E,), N, jnp.int32)])
    ea = jnp.concatenate([edge_attr, jnp.zeros((EP - E, 3), jnp.float32)])

    toff = (jnp.arange(T, dtype=jnp.int32) * NP)[:, None]
    rows_s = (toff + srcp[None, :]).reshape(T, NW, CPW, 128)
    rows_d = (toff + dstp[None, :]).reshape(T, NW, CPW, 128)
    dst_tiled = dstp.reshape(NW, CPW, 128)
    z32 = jnp.zeros((RPT, 32), jnp.float32)
    z8 = jnp.zeros((RPT, 8), jnp.float32)

    H = jnp.zeros((T, NP, 4), jnp.float32).at[:, :N, :].set(
        jnp.swapaxes(x, 0, 1))

    def layer(H, Wl, Wr, We, a, b):
        F = H.shape[-1]
        XL = _mm(H.reshape(T * NP, F), Wl, 5008)
        XR = _mm(H.reshape(T * NP, F), Wr, 5008)
        EE = _mm(ea, We, BE)
        n0, n1, d0, d1 = _sc_edge_layer(XL, XR, EE, rows_s, rows_d,
                                        dst_tiled,
                                        jnp.tile(a.reshape(-1)[:, None], (1, 16)),
                                        z32, z8)
        return _finalize(n0, n1, d0, d1, Exp, b.reshape(1, 32))

    H1 = layer(H, p['Wl1'], p['Wr1'], p['We1'], p['a1'], p['b1'])
    H2 = layer(H1, p['Wl2'], p['Wr2'], p['We2'], p['a2'], p['b2'])

    Mp = jnp.zeros((NP, 128), jnp.float32).at[:N, :100].set(M)
    G = _gen_einsum(Mp, H2)
    Wo = p['Wo']
    logits = _decoder(G, p['Wih_f'], p['Whh_f'], p['bf'].reshape(1, 128),
                      p['Wih_b'], p['Whh_b'], p['bb'].reshape(1, 128),
                      Wo[:32, 0].reshape(1, 32), Wo[32:, 0].reshape(1, 32),
                      p['bo'].reshape(1, 1))
    return jnp.swapaxes(logits, 0, 1)[:100, :]


# speedup vs baseline: 13.3625x; 1.1566x over previous
import functools

import jax
import jax.numpy as jnp
import numpy as np
from jax import lax
from jax.experimental import pallas as pl
from jax.experimental.pallas import tpu as pltpu
from jax.experimental.pallas import tpu_sc as plsc

N = 10000
NP = 10016
E = 160000
EP = 163840
T = 24
DH = 32
NW = 32
CPW = EP // NW // 128
EPW = EP // NW
RPT = NP // 16

BE = 2048
BN = 2504


def _expand8():
    m = np.zeros((8, 32), np.float32)
    for h in range(4):
        m[h, h * 8:(h + 1) * 8] = 1.0
    return jnp.asarray(m)


def _amat8(a):
    z = jnp.zeros((32, 8), jnp.float32)
    for h in range(4):
        z = z.at[h * 8:(h + 1) * 8, h].set(a[h])
    return z


def _mm(A, W, bm):
    M, K = A.shape
    D = W.shape[1]

    def body(a_ref, w_ref, o_ref):
        o_ref[...] = lax.dot_general(a_ref[...], w_ref[...],
                                     (((1,), (0,)), ((), ())),
                                     preferred_element_type=jnp.float32, precision=lax.Precision.HIGHEST)

    return pl.pallas_call(
        body,
        grid=(M // bm,),
        in_specs=[pl.BlockSpec((bm, K), lambda i: (i, 0)),
                  pl.BlockSpec((K, D), lambda i: (0, 0))],
        out_specs=pl.BlockSpec((bm, D), lambda i: (i, 0)),
        out_shape=jax.ShapeDtypeStruct((M, D), jnp.float32),
    )(A, W)


def _edge_math(S, R, EE, Amat, Exp, mask):
    def body(s_ref, r_ref, ee_ref, am_ref, ex_ref, mk_ref, wm_o, e8_o):
        u = s_ref[0] + r_ref[0] + ee_ref[...]
        u = jnp.where(u >= 0, u, 0.2 * u)
        logit = lax.dot_general(u, am_ref[...], (((1,), (0,)), ((), ())),
                                preferred_element_type=jnp.float32, precision=lax.Precision.HIGHEST)
        ex = jnp.exp(logit) * mk_ref[...]
        e8_o[0] = ex
        wm_o[0] = lax.dot_general(ex, ex_ref[...], (((1,), (0,)), ((), ())),
                                  preferred_element_type=jnp.float32, precision=lax.Precision.HIGHEST) * s_ref[0]

    return pl.pallas_call(
        body,
        grid=(T, EP // BE),
        in_specs=[
            pl.BlockSpec((1, BE, 32), lambda t, e: (t, e, 0)),
            pl.BlockSpec((1, BE, 32), lambda t, e: (t, e, 0)),
            pl.BlockSpec((BE, 32), lambda t, e: (e, 0)),
            pl.BlockSpec((32, 8), lambda t, e: (0, 0)),
            pl.BlockSpec((8, 32), lambda t, e: (0, 0)),
            pl.BlockSpec((1, 8), lambda t, e: (0, 0)),
        ],
        out_specs=[
            pl.BlockSpec((1, BE, 32), lambda t, e: (t, e, 0)),
            pl.BlockSpec((1, BE, 8), lambda t, e: (t, e, 0)),
        ],
        out_shape=[
            jax.ShapeDtypeStruct((T, EP, 32), jnp.float32),
            jax.ShapeDtypeStruct((T, EP, 8), jnp.float32),
        ],
    )(S, R, EE, Amat, Exp, mask)


def _finalize(n0, n1, d0, d1, Exp, b):
    def body(n0_r, n1_r, d0_r, d1_r, ex_r, b_r, o_r):
        num = n0_r[0] + n1_r[0]
        den = lax.dot_general(d0_r[0] + d1_r[0], ex_r[...],
                              (((1,), (0,)), ((), ())),
                              preferred_element_type=jnp.float32, precision=lax.Precision.HIGHEST) + 1e-16
        h = num / den + b_r[...]
        o_r[0] = jnp.where(h > 0, h, jnp.exp(h) - 1.0)

    return pl.pallas_call(
        body,
        grid=(T, NP // BN),
        in_specs=[
            pl.BlockSpec((1, BN, 32), lambda t, n: (t, n, 0)),
            pl.BlockSpec((1, BN, 32), lambda t, n: (t, n, 0)),
            pl.BlockSpec((1, BN, 8), lambda t, n: (t, n, 0)),
            pl.BlockSpec((1, BN, 8), lambda t, n: (t, n, 0)),
            pl.BlockSpec((8, 32), lambda t, n: (0, 0)),
            pl.BlockSpec((1, 32), lambda t, n: (0, 0)),
        ],
        out_specs=pl.BlockSpec((1, BN, 32), lambda t, n: (t, n, 0)),
        out_shape=jax.ShapeDtypeStruct((T, NP, 32), jnp.float32),
    )(n0, n1, d0, d1, Exp, b)


def _gen_einsum(Mp, H2):
    def body(m_ref, q_ref, o_ref):
        @pl.when(pl.program_id(1) == 0)
        def _():
            o_ref[...] = jnp.zeros_like(o_ref)

        o_ref[0] += lax.dot_general(m_ref[...], q_ref[0],
                                    (((0,), (0,)), ((), ())),
                                    preferred_element_type=jnp.float32, precision=lax.Precision.HIGHEST)

    return pl.pallas_call(
        body,
        grid=(T, NP // BN),
        in_specs=[
            pl.BlockSpec((BN, 128), lambda t, n: (n, 0)),
            pl.BlockSpec((1, BN, 32), lambda t, n: (t, n, 0)),
        ],
        out_specs=pl.BlockSpec((1, 128, 32), lambda t, n: (t, 0, 0)),
        out_shape=jax.ShapeDtypeStruct((T, 128, 32), jnp.float32),
    )(Mp, H2)


def _decoder(G, Wih_f, Whh_f, bf, Wih_b, Whh_b, bb, wof, wob, bo):
    def sig(x):
        return 1.0 / (1.0 + jnp.exp(-x))

    def body(g_r, wif_r, whf_r, bf_r, wib_r, whb_r, bb_r, wof_r, wob_r,
             bo_r, o_r):
        def cell(xt, h, c, wi, wh, bias):
            z = (lax.dot_general(xt, wi, (((1,), (1,)), ((), ())),
                                 preferred_element_type=jnp.float32, precision=lax.Precision.HIGHEST)
                 + lax.dot_general(h, wh, (((1,), (1,)), ((), ())),
                                   preferred_element_type=jnp.float32, precision=lax.Precision.HIGHEST)
                 + bias)
            i = z[:, 0:32]
            f = z[:, 32:64]
            g = z[:, 64:96]
            o = z[:, 96:128]
            c = sig(f) * c + sig(i) * jnp.tanh(g)
            h = sig(o) * jnp.tanh(c)
            return h, c

        zero = jnp.zeros((128, 32), jnp.float32)
        h, c = zero, zero
        hf = []
        for t in range(T):
            h, c = cell(g_r[t], h, c, wif_r[...], whf_r[...], bf_r[...])
            hf.append(h)
        h, c = zero, zero
        for t in range(T - 1, -1, -1):
            h, c = cell(g_r[t], h, c, wib_r[...], whb_r[...], bb_r[...])
            logit = (jnp.sum(hf[t] * wof_r[...], axis=1)
                     + jnp.sum(h * wob_r[...], axis=1) + bo_r[0, 0])
            o_r[t, :] = logit

    full = lambda s: pl.BlockSpec(s, lambda: tuple(0 for _ in s))
    return pl.pallas_call(
        body,
        in_specs=[
            full((T, 128, 32)),
            full((128, 32)), full((128, 32)), full((1, 128)),
            full((128, 32)), full((128, 32)), full((1, 128)),
            full((1, 32)), full((1, 32)), full((1, 1)),
        ],
        out_specs=full((T, 128)),
        out_shape=jax.ShapeDtypeStruct((T, 128), jnp.float32),
    )(G, Wih_f, Whh_f, bf, Wih_b, Whh_b, bb, wof, wob, bo)


_MESH = plsc.VectorSubcoreMesh(core_axis_name="c", subcore_axis_name="s")


def _sc_edge_layer(XL, XR, EE, rows_s, rows_d, dst_tiled, a_flat, z32, z8):
    @functools.partial(
        pl.kernel, mesh=_MESH,
        compiler_params=pltpu.CompilerParams(use_tc_tiling_on_sc=False,
                                             needs_layout_passes=False),
        out_type=(jax.ShapeDtypeStruct((2 * T, NP, 32), jnp.float32),
                  jax.ShapeDtypeStruct((2 * T, NP, 8), jnp.float32)),
        scratch_types=[
            pltpu.VMEM((CPW, 128), jnp.int32),
            pltpu.VMEM((CPW, 128), jnp.int32),
            pltpu.VMEM((CPW, 128), jnp.int32),
            pltpu.VMEM((32, 16), jnp.float32),
            pltpu.VMEM((128, 32), jnp.float32),
            pltpu.VMEM((128, 32), jnp.float32),
            pltpu.VMEM((128, 32), jnp.float32),
            pltpu.VMEM((128, 32), jnp.float32),
            pltpu.VMEM((128, 32), jnp.float32),
            pltpu.VMEM((128, 32), jnp.float32),
            pltpu.VMEM((128, 32), jnp.float32),
            pltpu.VMEM((128, 8), jnp.float32),
            pltpu.VMEM_SHARED((NP, 32), jnp.float32),
            pltpu.VMEM_SHARED((NP, 8), jnp.float32),
            pltpu.SemaphoreType.DMA,
            pltpu.SemaphoreType.DMA,
        ],
    )
    def k(xl_hbm, xr_hbm, ee_hbm, is_hbm, id_hbm, dt_hbm, a_hbm, z32_hbm,
          z8_hbm, num_out, den_out,
          is_v, id_v, dv, a_v, sA, rA, eA, sB, rB, eB, wmbuf, exbuf,
          acc_n, acc_d, semA, semB):
        iota = lax.iota(jnp.int32, 16)
        cid = lax.axis_index("c")
        sid = lax.axis_index("s")
        wid = sid * 2 + cid
        ebase = wid * EPW
        rbase = sid * RPT

        pltpu.sync_copy(dt_hbm.at[wid], dv)
        pltpu.sync_copy(a_hbm, a_v)
        pltpu.sync_copy(z8_hbm.at[pl.ds(0, 128)], exbuf)

        def start(kk, sX, rX, eX, sem):
            pltpu.async_copy(xl_hbm.at[is_v.at[kk]], sX, sem)
            pltpu.async_copy(xr_hbm.at[id_v.at[kk]], rX, sem)
            pltpu.async_copy(ee_hbm.at[pl.ds(ebase + kk * 128, 128)], eX, sem)

        def waitbufs(sX, rX, eX, sem):
            pltpu.make_async_copy(xl_hbm.at[is_v.at[0]], sX, sem).wait()
            pltpu.make_async_copy(xr_hbm.at[id_v.at[0]], rX, sem).wait()
            pltpu.make_async_copy(
                ee_hbm.at[pl.ds(ebase, 128)], eX, sem).wait()

        def exp32(xv):
            nf = (xv * 1.4426950408889634 + 12582912.0) - 12582912.0
            r = (xv - nf * 0.693359375) + nf * 2.1219444005469057e-4
            p = 1.0 / 720.0 + r * 0.0
            p = 1.0 / 120.0 + r * p
            p = 1.0 / 24.0 + r * p
            p = 1.0 / 6.0 + r * p
            p = 0.5 + r * p
            p = 1.0 + r * p
            p = 1.0 + r * p
            ni = lax.convert_element_type(nf, jnp.int32)
            sc = lax.bitcast_convert_type(
                lax.shift_left(ni + 127, 23), jnp.float32)
            return p * sc

        def compute_scatter(kk, sX, rX, eX):
            def grp(g, c3):
                jvec = iota + g * 16
                acc = [jnp.zeros((16,), jnp.float32) for _ in range(4)]
                for c in range(32):
                    cv = jnp.full((16,), c, jnp.int32)
                    u = (plsc.load_gather(sX, [jvec, cv])
                         + plsc.load_gather(rX, [jvec, cv])
                         + plsc.load_gather(eX, [jvec, cv]))
                    u = jnp.maximum(u, 0.0) + 0.2 * jnp.minimum(u, 0.0)
                    acc[c // 8] = acc[c // 8] + u * a_v[c, :]
                for h in range(4):
                    hv = jnp.full((16,), h, jnp.int32)
                    exh = exp32(acc[h])
                    plsc.store_scatter(exbuf, [jvec, hv], exh)
                    for d in range(8):
                        c = h * 8 + d
                        cv = jnp.full((16,), c, jnp.int32)
                        wc = plsc.load_gather(sX, [jvec, cv]) * exh
                        plsc.store_scatter(wmbuf, [jvec, cv], wc)
                return c3

            lax.fori_loop(0, 8, grp, 0)
            pltpu.sync_copy(wmbuf, acc_n.at[dv.at[kk]], add=True)
            pltpu.sync_copy(exbuf, acc_d.at[dv.at[kk]], add=True)

        def per_t(t, carry):
            pltpu.sync_copy(is_hbm.at[t, wid], is_v)
            pltpu.sync_copy(id_hbm.at[t, wid], id_v)
            pltpu.sync_copy(z32_hbm, acc_n.at[pl.ds(rbase, RPT)])
            pltpu.sync_copy(z8_hbm, acc_d.at[pl.ds(rbase, RPT)])
            plsc.subcore_barrier()

            start(0, sA, rA, eA, semA)

            def pair(g2, c2):
                k0 = 2 * g2
                start(k0 + 1, sB, rB, eB, semB)
                waitbufs(sA, rA, eA, semA)
                compute_scatter(k0, sA, rA, eA)

                @pl.when(g2 + 1 < CPW // 2)
                def _():
                    start(k0 + 2, sA, rA, eA, semA)

                waitbufs(sB, rB, eB, semB)
                compute_scatter(k0 + 1, sB, rB, eB)
                return c2

            lax.fori_loop(0, CPW // 2, pair, 0)

            plsc.subcore_barrier()
            row = cid * T + t
            pltpu.sync_copy(acc_n.at[pl.ds(rbase, RPT)],
                            num_out.at[row, pl.ds(rbase, RPT)])
            pltpu.sync_copy(acc_d.at[pl.ds(rbase, RPT)],
                            den_out.at[row, pl.ds(rbase, RPT)])
            plsc.subcore_barrier()
            return carry

        lax.fori_loop(0, T, per_t, 0)

    num, den = k(XL, XR, EE, rows_s, rows_d, dst_tiled, a_flat, z32, z8)
    num = num.reshape(2, T, NP, 32)
    den = den.reshape(2, T, NP, 8)
    return num[0], num[1], den[0], den[1]


def _sc_gather(XL, XR, rows_s, rows_d):
    GC = 8
    GB = GC * 128

    @functools.partial(
        pl.kernel, mesh=_MESH,
        compiler_params=pltpu.CompilerParams(use_tc_tiling_on_sc=False),
        out_type=(jax.ShapeDtypeStruct((T, EP, 32), jnp.float32),
                  jax.ShapeDtypeStruct((T, EP, 32), jnp.float32)),
        scratch_types=[
            pltpu.VMEM((CPW, 128), jnp.int32),
            pltpu.VMEM((CPW, 128), jnp.int32),
            pltpu.VMEM((GB, 32), jnp.float32),
            pltpu.SemaphoreType.DMA,
        ],
    )
    def k(xl_hbm, xr_hbm, is_hbm, id_hbm, s_out, r_out, is_v, id_v, buf, sem):
        wid = lax.axis_index("s") * 2 + lax.axis_index("c")
        ebase = wid * EPW

        def per_t(t, carry):
            pltpu.sync_copy(is_hbm.at[t, wid], is_v)
            pltpu.sync_copy(id_hbm.at[t, wid], id_v)

            def make_pass(table, out, idxv):
                def grp(g, c2):
                    hs = [pltpu.async_copy(table.at[idxv.at[g * GC + j]],
                                           buf.at[pl.ds(j * 128, 128)], sem)
                          for j in range(GC)]
                    for h in hs:
                        h.wait()
                    pltpu.sync_copy(buf, out.at[t, pl.ds(ebase + g * GB, GB)])
                    return c2

                lax.fori_loop(0, CPW // GC, grp, 0)

            make_pass(xl_hbm, s_out, is_v)
            make_pass(xr_hbm, r_out, id_v)
            return carry

        lax.fori_loop(0, T, per_t, 0)

    return k(XL, XR, rows_s, rows_d)


def _sc_scatter(wm, ex8, dst_tiled, z32, z8):
    GC = 4
    GB = GC * 128

    @functools.partial(
        pl.kernel, mesh=_MESH,
        compiler_params=pltpu.CompilerParams(use_tc_tiling_on_sc=False),
        out_type=(jax.ShapeDtypeStruct((2 * T, NP, 32), jnp.float32),
                  jax.ShapeDtypeStruct((2 * T, NP, 8), jnp.float32)),
        scratch_types=[
            pltpu.VMEM((CPW, 128), jnp.int32),
            pltpu.VMEM((GB, 32), jnp.float32),
            pltpu.VMEM((GB, 8), jnp.float32),
            pltpu.VMEM_SHARED((NP, 32), jnp.float32),
            pltpu.VMEM_SHARED((NP, 8), jnp.float32),
            pltpu.SemaphoreType.DMA,
        ],
    )
    def k(wm_hbm, e8_hbm, dt_hbm, z32_hbm, z8_hbm, num_out, den_out,
          dv, wbuf, ebuf, acc_n, acc_d, sem):
        cid = lax.axis_index("c")
        sid = lax.axis_index("s")
        wid = sid * 2 + cid
        ebase = wid * EPW
        rbase = sid * RPT

        pltpu.sync_copy(dt_hbm.at[wid], dv)

        def per_t(t, carry):
            pltpu.sync_copy(z32_hbm, acc_n.at[pl.ds(rbase, RPT)])
            pltpu.sync_copy(z8_hbm, acc_d.at[pl.ds(rbase, RPT)])
            plsc.subcore_barrier()

            def grp(g, c2):
                h1 = pltpu.async_copy(
                    wm_hbm.at[t, pl.ds(ebase + g * GB, GB)], wbuf, sem)
                h2 = pltpu.async_copy(
                    e8_hbm.at[t, pl.ds(ebase + g * GB, GB)], ebuf, sem)
                h1.wait()
                h2.wait()
                for j in range(GC):
                    pltpu.sync_copy(wbuf.at[pl.ds(j * 128, 128)],
                                    acc_n.at[dv.at[g * GC + j]], add=True)
                    pltpu.sync_copy(ebuf.at[pl.ds(j * 128, 128)],
                                    acc_d.at[dv.at[g * GC + j]], add=True)
                return c2

            lax.fori_loop(0, CPW // GC, grp, 0)

            plsc.subcore_barrier()
            row = cid * T + t
            pltpu.sync_copy(acc_n.at[pl.ds(rbase, RPT)],
                            num_out.at[row, pl.ds(rbase, RPT)])
            pltpu.sync_copy(acc_d.at[pl.ds(rbase, RPT)],
                            den_out.at[row, pl.ds(rbase, RPT)])
            plsc.subcore_barrier()
            return carry

        lax.fori_loop(0, T, per_t, 0)

    num, den = k(wm, ex8, dst_tiled, z32, z8)
    num = num.reshape(2, T, NP, 32)
    den = den.reshape(2, T, NP, 8)
    return num[0], num[1], den[0], den[1]


def kernel(x, edge_index, edge_attr, M, params):
    p = params
    Exp = _expand8()
    mask = jnp.asarray([[1.0] * 4 + [0.0] * 4], jnp.float32)

    srcp = jnp.concatenate([edge_index[0], jnp.zeros((EP - E,), jnp.int32)])
    dstp = jnp.concatenate([edge_index[1], jnp.full((EP - E,), N, jnp.int32)])
    ea = jnp.concatenate([edge_attr, jnp.zeros((EP - E, 3), jnp.float32)])

    toff = (jnp.arange(T, dtype=jnp.int32) * NP)[:, None]
    rows_s = (toff + srcp[None, :]).reshape(T, NW, CPW, 128)
    rows_d = (toff + dstp[None, :]).reshape(T, NW, CPW, 128)
    dst_tiled = dstp.reshape(NW, CPW, 128)
    z32 = jnp.zeros((RPT, 32), jnp.float32)
    z8 = jnp.zeros((RPT, 8), jnp.float32)

    H = jnp.zeros((T, NP, 4), jnp.float32).at[:, :N, :].set(
        jnp.swapaxes(x, 0, 1))

    def layer(H, Wl, Wr, We, a, b):
        F = H.shape[-1]
        XL = _mm(H.reshape(T * NP, F), Wl, 5008)
        XR = _mm(H.reshape(T * NP, F), Wr, 5008)
        EE = _mm(ea, We, BE)
        n0, n1, d0, d1 = _sc_edge_layer(XL, XR, EE, rows_s, rows_d,
                                        dst_tiled,
                                        jnp.tile(a.reshape(-1)[:, None], (1, 16)),
                                        z32, z8)
        return _finalize(n0, n1, d0, d1, Exp, b.reshape(1, 32))

    H1 = layer(H, p['Wl1'], p['Wr1'], p['We1'], p['a1'], p['b1'])
    H2 = layer(H1, p['Wl2'], p['Wr2'], p['We2'], p['a2'], p['b2'])

    Mp = jnp.zeros((NP, 128), jnp.float32).at[:N, :100].set(M)
    G = _gen_einsum(Mp, H2)
    Wo = p['Wo']
    logits = _decoder(G, p['Wih_f'], p['Whh_f'], p['bf'].reshape(1, 128),
                      p['Wih_b'], p['Whh_b'], p['bb'].reshape(1, 128),
                      Wo[:32, 0].reshape(1, 32), Wo[32:, 0].reshape(1, 32),
                      p['bo'].reshape(1, 1))
    return jnp.swapaxes(logits, 0, 1)[:100, :]


# speedup vs baseline: 13.6726x; 1.0232x over previous
import functools

import jax
import jax.numpy as jnp
import numpy as np
from jax import lax
from jax.experimental import pallas as pl
from jax.experimental.pallas import tpu as pltpu
from jax.experimental.pallas import tpu_sc as plsc

N = 10000
NP = 10016
E = 160000
EP = 163840
T = 24
DH = 32
NW = 32
CPW = EP // NW // 128
EPW = EP // NW
RPT = NP // 16

BE = 2048
BN = 2504


def _expand8():
    m = np.zeros((8, 32), np.float32)
    for h in range(4):
        m[h, h * 8:(h + 1) * 8] = 1.0
    return jnp.asarray(m)


def _amat8(a):
    z = jnp.zeros((32, 8), jnp.float32)
    for h in range(4):
        z = z.at[h * 8:(h + 1) * 8, h].set(a[h])
    return z


def _mm(A, W, bm):
    M, K = A.shape
    D = W.shape[1]

    def body(a_ref, w_ref, o_ref):
        o_ref[...] = lax.dot_general(a_ref[...], w_ref[...],
                                     (((1,), (0,)), ((), ())),
                                     preferred_element_type=jnp.float32, precision=lax.Precision.HIGHEST)

    return pl.pallas_call(
        body,
        grid=(M // bm,),
        in_specs=[pl.BlockSpec((bm, K), lambda i: (i, 0)),
                  pl.BlockSpec((K, D), lambda i: (0, 0))],
        out_specs=pl.BlockSpec((bm, D), lambda i: (i, 0)),
        out_shape=jax.ShapeDtypeStruct((M, D), jnp.float32),
    )(A, W)


def _edge_math(S, R, EE, Amat, Exp, mask):
    def body(s_ref, r_ref, ee_ref, am_ref, ex_ref, mk_ref, wm_o, e8_o):
        u = s_ref[0] + r_ref[0] + ee_ref[...]
        u = jnp.where(u >= 0, u, 0.2 * u)
        logit = lax.dot_general(u, am_ref[...], (((1,), (0,)), ((), ())),
                                preferred_element_type=jnp.float32, precision=lax.Precision.HIGHEST)
        ex = jnp.exp(logit) * mk_ref[...]
        e8_o[0] = ex
        wm_o[0] = lax.dot_general(ex, ex_ref[...], (((1,), (0,)), ((), ())),
                                  preferred_element_type=jnp.float32, precision=lax.Precision.HIGHEST) * s_ref[0]

    return pl.pallas_call(
        body,
        grid=(T, EP // BE),
        in_specs=[
            pl.BlockSpec((1, BE, 32), lambda t, e: (t, e, 0)),
            pl.BlockSpec((1, BE, 32), lambda t, e: (t, e, 0)),
            pl.BlockSpec((BE, 32), lambda t, e: (e, 0)),
            pl.BlockSpec((32, 8), lambda t, e: (0, 0)),
            pl.BlockSpec((8, 32), lambda t, e: (0, 0)),
            pl.BlockSpec((1, 8), lambda t, e: (0, 0)),
        ],
        out_specs=[
            pl.BlockSpec((1, BE, 32), lambda t, e: (t, e, 0)),
            pl.BlockSpec((1, BE, 8), lambda t, e: (t, e, 0)),
        ],
        out_shape=[
            jax.ShapeDtypeStruct((T, EP, 32), jnp.float32),
            jax.ShapeDtypeStruct((T, EP, 8), jnp.float32),
        ],
    )(S, R, EE, Amat, Exp, mask)


def _finalize(n0, n1, d0, d1, Exp, b):
    def body(n0_r, n1_r, d0_r, d1_r, ex_r, b_r, o_r):
        num = n0_r[0] + n1_r[0]
        den = lax.dot_general(d0_r[0] + d1_r[0], ex_r[...],
                              (((1,), (0,)), ((), ())),
                              preferred_element_type=jnp.float32, precision=lax.Precision.HIGHEST) + 1e-16
        h = num / den + b_r[...]
        o_r[0] = jnp.where(h > 0, h, jnp.exp(h) - 1.0)

    return pl.pallas_call(
        body,
        grid=(T, NP // BN),
        in_specs=[
            pl.BlockSpec((1, BN, 32), lambda t, n: (t, n, 0)),
            pl.BlockSpec((1, BN, 32), lambda t, n: (t, n, 0)),
            pl.BlockSpec((1, BN, 8), lambda t, n: (t, n, 0)),
            pl.BlockSpec((1, BN, 8), lambda t, n: (t, n, 0)),
            pl.BlockSpec((8, 32), lambda t, n: (0, 0)),
            pl.BlockSpec((1, 32), lambda t, n: (0, 0)),
        ],
        out_specs=pl.BlockSpec((1, BN, 32), lambda t, n: (t, n, 0)),
        out_shape=jax.ShapeDtypeStruct((T, NP, 32), jnp.float32),
    )(n0, n1, d0, d1, Exp, b)


def _gen_einsum(Mp, H2):
    def body(m_ref, q_ref, o_ref):
        @pl.when(pl.program_id(1) == 0)
        def _():
            o_ref[...] = jnp.zeros_like(o_ref)

        o_ref[0] += lax.dot_general(m_ref[...], q_ref[0],
                                    (((0,), (0,)), ((), ())),
                                    preferred_element_type=jnp.float32, precision=lax.Precision.HIGHEST)

    return pl.pallas_call(
        body,
        grid=(T, NP // BN),
        in_specs=[
            pl.BlockSpec((BN, 128), lambda t, n: (n, 0)),
            pl.BlockSpec((1, BN, 32), lambda t, n: (t, n, 0)),
        ],
        out_specs=pl.BlockSpec((1, 128, 32), lambda t, n: (t, 0, 0)),
        out_shape=jax.ShapeDtypeStruct((T, 128, 32), jnp.float32),
    )(Mp, H2)


def _decoder(G, Wih_f, Whh_f, bf, Wih_b, Whh_b, bb, wof, wob, bo):
    def sig(x):
        return 1.0 / (1.0 + jnp.exp(-x))

    def body(g_r, wif_r, whf_r, bf_r, wib_r, whb_r, bb_r, wof_r, wob_r,
             bo_r, o_r):
        def cell(xt, h, c, wi, wh, bias):
            z = (lax.dot_general(xt, wi, (((1,), (1,)), ((), ())),
                                 preferred_element_type=jnp.float32, precision=lax.Precision.HIGHEST)
                 + lax.dot_general(h, wh, (((1,), (1,)), ((), ())),
                                   preferred_element_type=jnp.float32, precision=lax.Precision.HIGHEST)
                 + bias)
            i = z[:, 0:32]
            f = z[:, 32:64]
            g = z[:, 64:96]
            o = z[:, 96:128]
            c = sig(f) * c + sig(i) * jnp.tanh(g)
            h = sig(o) * jnp.tanh(c)
            return h, c

        zero = jnp.zeros((128, 32), jnp.float32)
        h, c = zero, zero
        hf = []
        for t in range(T):
            h, c = cell(g_r[t], h, c, wif_r[...], whf_r[...], bf_r[...])
            hf.append(h)
        h, c = zero, zero
        for t in range(T - 1, -1, -1):
            h, c = cell(g_r[t], h, c, wib_r[...], whb_r[...], bb_r[...])
            logit = (jnp.sum(hf[t] * wof_r[...], axis=1)
                     + jnp.sum(h * wob_r[...], axis=1) + bo_r[0, 0])
            o_r[t, :] = logit

    full = lambda s: pl.BlockSpec(s, lambda: tuple(0 for _ in s))
    return pl.pallas_call(
        body,
        in_specs=[
            full((T, 128, 32)),
            full((128, 32)), full((128, 32)), full((1, 128)),
            full((128, 32)), full((128, 32)), full((1, 128)),
            full((1, 32)), full((1, 32)), full((1, 1)),
        ],
        out_specs=full((T, 128)),
        out_shape=jax.ShapeDtypeStruct((T, 128), jnp.float32),
    )(G, Wih_f, Whh_f, bf, Wih_b, Whh_b, bb, wof, wob, bo)


_MESH = plsc.VectorSubcoreMesh(core_axis_name="c", subcore_axis_name="s")


def _sc_edge_layer(XL, XR, EE, rows_s, rows_d, dst_tiled, a_flat, z32, z8):
    @functools.partial(
        pl.kernel, mesh=_MESH,
        compiler_params=pltpu.CompilerParams(use_tc_tiling_on_sc=False,
                                             needs_layout_passes=False),
        out_type=(jax.ShapeDtypeStruct((2 * T, NP, 32), jnp.float32),
                  jax.ShapeDtypeStruct((2 * T, NP, 8), jnp.float32)),
        scratch_types=[
            pltpu.VMEM((CPW, 128), jnp.int32),
            pltpu.VMEM((CPW, 128), jnp.int32),
            pltpu.VMEM((CPW, 128), jnp.int32),
            pltpu.VMEM((32, 16), jnp.float32),
            pltpu.VMEM((128, 32), jnp.float32),
            pltpu.VMEM((128, 32), jnp.float32),
            pltpu.VMEM((128, 32), jnp.float32),
            pltpu.VMEM((128, 32), jnp.float32),
            pltpu.VMEM((128, 32), jnp.float32),
            pltpu.VMEM((128, 32), jnp.float32),
            pltpu.VMEM((128, 32), jnp.float32),
            pltpu.VMEM((128, 32), jnp.float32),
            pltpu.VMEM((128, 8), jnp.float32),
            pltpu.VMEM((128, 8), jnp.float32),
            pltpu.VMEM_SHARED((NP, 32), jnp.float32),
            pltpu.VMEM_SHARED((NP, 8), jnp.float32),
            pltpu.SemaphoreType.DMA,
            pltpu.SemaphoreType.DMA,
            pltpu.SemaphoreType.DMA,
            pltpu.SemaphoreType.DMA,
        ],
    )
    def k(xl_hbm, xr_hbm, ee_hbm, is_hbm, id_hbm, dt_hbm, a_hbm, z32_hbm,
          z8_hbm, num_out, den_out,
          is_v, id_v, dv, a_v, sA, rA, eA, sB, rB, eB, wmA, wmB, exA, exB,
          acc_n, acc_d, semA, semB, semWA, semWB):
        iota = lax.iota(jnp.int32, 16)
        cid = lax.axis_index("c")
        sid = lax.axis_index("s")
        wid = sid * 2 + cid
        ebase = wid * EPW
        rbase = sid * RPT

        pltpu.sync_copy(dt_hbm.at[wid], dv)
        pltpu.sync_copy(a_hbm, a_v)
        pltpu.sync_copy(z8_hbm.at[pl.ds(0, 128)], exA)
        pltpu.sync_copy(z8_hbm.at[pl.ds(0, 128)], exB)

        def start(kk, sX, rX, eX, sem):
            pltpu.async_copy(xl_hbm.at[is_v.at[kk]], sX, sem)
            pltpu.async_copy(xr_hbm.at[id_v.at[kk]], rX, sem)
            pltpu.async_copy(ee_hbm.at[pl.ds(ebase + kk * 128, 128)], eX, sem)

        def waitbufs(sX, rX, eX, sem):
            pltpu.make_async_copy(xl_hbm.at[is_v.at[0]], sX, sem).wait()
            pltpu.make_async_copy(xr_hbm.at[id_v.at[0]], rX, sem).wait()
            pltpu.make_async_copy(
                ee_hbm.at[pl.ds(ebase, 128)], eX, sem).wait()

        def exp32(xv):
            nf = (xv * 1.4426950408889634 + 12582912.0) - 12582912.0
            r = (xv - nf * 0.693359375) + nf * 2.1219444005469057e-4
            p = 1.0 / 720.0 + r * 0.0
            p = 1.0 / 120.0 + r * p
            p = 1.0 / 24.0 + r * p
            p = 1.0 / 6.0 + r * p
            p = 0.5 + r * p
            p = 1.0 + r * p
            p = 1.0 + r * p
            ni = lax.convert_element_type(nf, jnp.int32)
            sc = lax.bitcast_convert_type(
                lax.shift_left(ni + 127, 23), jnp.float32)
            return p * sc

        def compute_scatter(kk, sX, rX, eX, wmbuf, exbuf, semW, do_wait):
            @pl.when(do_wait)
            def _():
                pltpu.make_async_copy(wmbuf, acc_n.at[dv.at[0]], semW).wait()
                pltpu.make_async_copy(exbuf, acc_d.at[dv.at[0]], semW).wait()

            def grp(g, c3):
                jvec = iota + g * 16
                acc = [jnp.zeros((16,), jnp.float32) for _ in range(4)]
                for c in range(32):
                    cv = jnp.full((16,), c, jnp.int32)
                    u = (plsc.load_gather(sX, [jvec, cv])
                         + plsc.load_gather(rX, [jvec, cv])
                         + plsc.load_gather(eX, [jvec, cv]))
                    u = jnp.maximum(u, 0.0) + 0.2 * jnp.minimum(u, 0.0)
                    acc[c // 8] = acc[c // 8] + u * a_v[c, :]
                for h in range(4):
                    hv = jnp.full((16,), h, jnp.int32)
                    exh = exp32(acc[h])
                    plsc.store_scatter(exbuf, [jvec, hv], exh)
                    for d in range(8):
                        c = h * 8 + d
                        cv = jnp.full((16,), c, jnp.int32)
                        wc = plsc.load_gather(sX, [jvec, cv]) * exh
                        plsc.store_scatter(wmbuf, [jvec, cv], wc)
                return c3

            lax.fori_loop(0, 8, grp, 0)
            pltpu.async_copy(wmbuf, acc_n.at[dv.at[kk]], semW, add=True)
            pltpu.async_copy(exbuf, acc_d.at[dv.at[kk]], semW, add=True)

        def per_t(t, carry):
            pltpu.sync_copy(is_hbm.at[t, wid], is_v)
            pltpu.sync_copy(id_hbm.at[t, wid], id_v)
            pltpu.sync_copy(z32_hbm, acc_n.at[pl.ds(rbase, RPT)])
            pltpu.sync_copy(z8_hbm, acc_d.at[pl.ds(rbase, RPT)])
            plsc.subcore_barrier()

            start(0, sA, rA, eA, semA)

            def pair(g2, c2):
                k0 = 2 * g2
                start(k0 + 1, sB, rB, eB, semB)
                waitbufs(sA, rA, eA, semA)
                compute_scatter(k0, sA, rA, eA, wmA, exA, semWA, g2 > 0)

                @pl.when(g2 + 1 < CPW // 2)
                def _():
                    start(k0 + 2, sA, rA, eA, semA)

                waitbufs(sB, rB, eB, semB)
                compute_scatter(k0 + 1, sB, rB, eB, wmB, exB, semWB, g2 > 0)
                return c2

            lax.fori_loop(0, CPW // 2, pair, 0)
            pltpu.make_async_copy(wmA, acc_n.at[dv.at[0]], semWA).wait()
            pltpu.make_async_copy(exA, acc_d.at[dv.at[0]], semWA).wait()
            pltpu.make_async_copy(wmB, acc_n.at[dv.at[0]], semWB).wait()
            pltpu.make_async_copy(exB, acc_d.at[dv.at[0]], semWB).wait()

            plsc.subcore_barrier()
            row = cid * T + t
            pltpu.sync_copy(acc_n.at[pl.ds(rbase, RPT)],
                            num_out.at[row, pl.ds(rbase, RPT)])
            pltpu.sync_copy(acc_d.at[pl.ds(rbase, RPT)],
                            den_out.at[row, pl.ds(rbase, RPT)])
            plsc.subcore_barrier()
            return carry

        lax.fori_loop(0, T, per_t, 0)

    num, den = k(XL, XR, EE, rows_s, rows_d, dst_tiled, a_flat, z32, z8)
    num = num.reshape(2, T, NP, 32)
    den = den.reshape(2, T, NP, 8)
    return num[0], num[1], den[0], den[1]


def _sc_gather(XL, XR, rows_s, rows_d):
    GC = 8
    GB = GC * 128

    @functools.partial(
        pl.kernel, mesh=_MESH,
        compiler_params=pltpu.CompilerParams(use_tc_tiling_on_sc=False),
        out_type=(jax.ShapeDtypeStruct((T, EP, 32), jnp.float32),
                  jax.ShapeDtypeStruct((T, EP, 32), jnp.float32)),
        scratch_types=[
            pltpu.VMEM((CPW, 128), jnp.int32),
            pltpu.VMEM((CPW, 128), jnp.int32),
            pltpu.VMEM((GB, 32), jnp.float32),
            pltpu.SemaphoreType.DMA,
        ],
    )
    def k(xl_hbm, xr_hbm, is_hbm, id_hbm, s_out, r_out, is_v, id_v, buf, sem):
        wid = lax.axis_index("s") * 2 + lax.axis_index("c")
        ebase = wid * EPW

        def per_t(t, carry):
            pltpu.sync_copy(is_hbm.at[t, wid], is_v)
            pltpu.sync_copy(id_hbm.at[t, wid], id_v)

            def make_pass(table, out, idxv):
                def grp(g, c2):
                    hs = [pltpu.async_copy(table.at[idxv.at[g * GC + j]],
                                           buf.at[pl.ds(j * 128, 128)], sem)
                          for j in range(GC)]
                    for h in hs:
                        h.wait()
                    pltpu.sync_copy(buf, out.at[t, pl.ds(ebase + g * GB, GB)])
                    return c2

                lax.fori_loop(0, CPW // GC, grp, 0)

            make_pass(xl_hbm, s_out, is_v)
            make_pass(xr_hbm, r_out, id_v)
            return carry

        lax.fori_loop(0, T, per_t, 0)

    return k(XL, XR, rows_s, rows_d)


def _sc_scatter(wm, ex8, dst_tiled, z32, z8):
    GC = 4
    GB = GC * 128

    @functools.partial(
        pl.kernel, mesh=_MESH,
        compiler_params=pltpu.CompilerParams(use_tc_tiling_on_sc=False),
        out_type=(jax.ShapeDtypeStruct((2 * T, NP, 32), jnp.float32),
                  jax.ShapeDtypeStruct((2 * T, NP, 8), jnp.float32)),
        scratch_types=[
            pltpu.VMEM((CPW, 128), jnp.int32),
            pltpu.VMEM((GB, 32), jnp.float32),
            pltpu.VMEM((GB, 8), jnp.float32),
            pltpu.VMEM_SHARED((NP, 32), jnp.float32),
            pltpu.VMEM_SHARED((NP, 8), jnp.float32),
            pltpu.SemaphoreType.DMA,
        ],
    )
    def k(wm_hbm, e8_hbm, dt_hbm, z32_hbm, z8_hbm, num_out, den_out,
          dv, wbuf, ebuf, acc_n, acc_d, sem):
        cid = lax.axis_index("c")
        sid = lax.axis_index("s")
        wid = sid * 2 + cid
        ebase = wid * EPW
        rbase = sid * RPT

        pltpu.sync_copy(dt_hbm.at[wid], dv)

        def per_t(t, carry):
            pltpu.sync_copy(z32_hbm, acc_n.at[pl.ds(rbase, RPT)])
            pltpu.sync_copy(z8_hbm, acc_d.at[pl.ds(rbase, RPT)])
            plsc.subcore_barrier()

            def grp(g, c2):
                h1 = pltpu.async_copy(
                    wm_hbm.at[t, pl.ds(ebase + g * GB, GB)], wbuf, sem)
                h2 = pltpu.async_copy(
                    e8_hbm.at[t, pl.ds(ebase + g * GB, GB)], ebuf, sem)
                h1.wait()
                h2.wait()
                for j in range(GC):
                    pltpu.sync_copy(wbuf.at[pl.ds(j * 128, 128)],
                                    acc_n.at[dv.at[g * GC + j]], add=True)
                    pltpu.sync_copy(ebuf.at[pl.ds(j * 128, 128)],
                                    acc_d.at[dv.at[g * GC + j]], add=True)
                return c2

            lax.fori_loop(0, CPW // GC, grp, 0)

            plsc.subcore_barrier()
            row = cid * T + t
            pltpu.sync_copy(acc_n.at[pl.ds(rbase, RPT)],
                            num_out.at[row, pl.ds(rbase, RPT)])
            pltpu.sync_copy(acc_d.at[pl.ds(rbase, RPT)],
                            den_out.at[row, pl.ds(rbase, RPT)])
            plsc.subcore_barrier()
            return carry

        lax.fori_loop(0, T, per_t, 0)

    num, den = k(wm, ex8, dst_tiled, z32, z8)
    num = num.reshape(2, T, NP, 32)
    den = den.reshape(2, T, NP, 8)
    return num[0], num[1], den[0], den[1]


def kernel(x, edge_index, edge_attr, M, params):
    p = params
    Exp = _expand8()
    mask = jnp.asarray([[1.0] * 4 + [0.0] * 4], jnp.float32)

    srcp = jnp.concatenate([edge_index[0], jnp.zeros((EP - E,), jnp.int32)])
    dstp = jnp.concatenate([edge_index[1], jnp.full((EP - E,), N, jnp.int32)])
    ea = jnp.concatenate([edge_attr, jnp.zeros((EP - E, 3), jnp.float32)])

    toff = (jnp.arange(T, dtype=jnp.int32) * NP)[:, None]
    rows_s = (toff + srcp[None, :]).reshape(T, NW, CPW, 128)
    rows_d = (toff + dstp[None, :]).reshape(T, NW, CPW, 128)
    dst_tiled = dstp.reshape(NW, CPW, 128)
    z32 = jnp.zeros((RPT, 32), jnp.float32)
    z8 = jnp.zeros((RPT, 8), jnp.float32)

    H = jnp.zeros((T, NP, 4), jnp.float32).at[:, :N, :].set(
        jnp.swapaxes(x, 0, 1))

    def layer(H, Wl, Wr, We, a, b):
        F = H.shape[-1]
        XL = _mm(H.reshape(T * NP, F), Wl, 5008)
        XR = _mm(H.reshape(T * NP, F), Wr, 5008)
        EE = _mm(ea, We, BE)
        n0, n1, d0, d1 = _sc_edge_layer(XL, XR, EE, rows_s, rows_d,
                                        dst_tiled,
                                        jnp.tile(a.reshape(-1)[:, None], (1, 16)),
                                        z32, z8)
        return _finalize(n0, n1, d0, d1, Exp, b.reshape(1, 32))

    H1 = layer(H, p['Wl1'], p['Wr1'], p['We1'], p['a1'], p['b1'])
    H2 = layer(H1, p['Wl2'], p['Wr2'], p['We2'], p['a2'], p['b2'])

    Mp = jnp.zeros((NP, 128), jnp.float32).at[:N, :100].set(M)
    G = _gen_einsum(Mp, H2)
    Wo = p['Wo']
    logits = _decoder(G, p['Wih_f'], p['Whh_f'], p['bf'].reshape(1, 128),
                      p['Wih_b'], p['Whh_b'], p['bb'].reshape(1, 128),
                      Wo[:32, 0].reshape(1, 32), Wo[32:, 0].reshape(1, 32),
                      p['bo'].reshape(1, 1))
    return jnp.swapaxes(logits, 0, 1)[:100, :]
